# baseline (speedup 1.0000x reference)
import functools

import jax
import jax.numpy as jnp
from jax import lax
from jax.experimental import pallas as pl
from jax.experimental.pallas import tpu as pltpu
from jax.experimental.pallas import tpu_sc as plsc

BATCH = 4096
HIST = 200
EMBED = 64
NC = 2
NS = 16
NW = NC * NS
LANES = 128
TT = HIST // 8


def _body(idx4_hbm, mask4_hbm, table_hbm, tw_hbm, out_hbm,
          idx_v, mask_v, rows0, rows1, twb0, twb1, out_acc, out_t,
          sem_r0, sem_r1, sem_t0, sem_t1):
  cid = lax.axis_index("c")
  sid = lax.axis_index("s")
  wid = sid * NC + cid

  pltpu.sync_copy(idx4_hbm.at[:, wid], idx_v)
  pltpu.sync_copy(mask4_hbm.at[:, wid], mask_v)

  rows_bufs = (rows0, rows1)
  tw_bufs = (twb0, twb1)
  sem_r = (sem_r0, sem_r1)
  sem_t = (sem_t0, sem_t1)

  z = jnp.zeros((16,), jnp.float32)

  def zero_body(i, carry):
    for k in range(EMBED // 16):
      out_acc[i, pl.ds(16 * k, 16)] = z
    return carry

  lax.fori_loop(0, LANES, zero_body, 0)

  def fire(tr, r, b):
    idxr = idx_v.at[tr, r]
    pltpu.async_copy(table_hbm.at[idxr], rows_bufs[b], sem_r[b])
    pltpu.async_copy(tw_hbm.at[idxr], tw_bufs[b], sem_t[b])

  def wait(b):
    pltpu.make_async_copy(table_hbm.at[pl.ds(0, LANES)], rows_bufs[b],
                          sem_r[b]).wait()
    pltpu.make_async_copy(tw_hbm.at[pl.ds(0, LANES)], tw_bufs[b],
                          sem_t[b]).wait()

  fire(0, 0, 0)
  fire(0, 1, 1)

  def outer(tr, carry):
    for r in range(8):
      b = r % 2
      wait(b)
      rows = rows_bufs[b]
      twb = tw_bufs[b]

      def chunk_body(c, carry2):
        m = mask_v[tr, r, pl.ds(16 * c, 16)]
        t = twb[pl.ds(16 * c, 16)]
        wv = m * m * t
        for u in range(16):
          i = 16 * c + u
          w = wv[u]
          for k in range(EMBED // 16):
            sl = pl.ds(16 * k, 16)
            plsc.addupdate(out_acc.at[i, sl], rows[i, sl] * w)
        return carry2

      lax.fori_loop(0, LANES // 16, chunk_body, 0)

      if r < 6:
        fire(tr, r + 2, b)
      else:
        @pl.when(tr + 1 < TT)
        def _():
          fire(tr + 1, r - 6, b)
    return carry

  lax.fori_loop(0, TT, outer, 0)

  iota = lax.iota(jnp.int32, 16)
  for c in range(LANES // 16):
    bidx = 16 * c + iota
    for k in range(EMBED):
      kidx = jnp.full((16,), k, jnp.int32)
      out_t[k // 8, k % 8, pl.ds(16 * c, 16)] = plsc.load_gather(
          out_acc, [bidx, kidx])

  pltpu.sync_copy(out_t, out_hbm.at[:, wid])


@functools.lru_cache(maxsize=1)
def _build():
  return functools.partial(
      pl.kernel,
      out_type=jax.ShapeDtypeStruct((EMBED // 8, NW, 8, LANES), jnp.float32),
      mesh=plsc.VectorSubcoreMesh(core_axis_name="c", subcore_axis_name="s"),
      scratch_types=[
          pltpu.VMEM((TT, 8, LANES), jnp.int32),
          pltpu.VMEM((TT, 8, LANES), jnp.float32),
          pltpu.VMEM((LANES, EMBED), jnp.float32),
          pltpu.VMEM((LANES, EMBED), jnp.float32),
          pltpu.VMEM((LANES,), jnp.float32),
          pltpu.VMEM((LANES,), jnp.float32),
          pltpu.VMEM((LANES, EMBED), jnp.float32),
          pltpu.VMEM((EMBED // 8, 8, LANES), jnp.float32),
          pltpu.SemaphoreType.DMA,
          pltpu.SemaphoreType.DMA,
          pltpu.SemaphoreType.DMA,
          pltpu.SemaphoreType.DMA,
      ],
      compiler_params=pltpu.CompilerParams(use_tc_tiling_on_sc=False,
                                           needs_layout_passes=False),
  )(_body)


def kernel(idxs, mask, table, token_weights):
  idx4 = idxs.astype(jnp.int32).reshape(32, 128, 25, 8).transpose(2, 0, 3, 1)
  mask4 = mask.reshape(32, 128, 25, 8).transpose(2, 0, 3, 1)
  out4 = _build()(idx4, mask4, table, token_weights)
  return out4.transpose(1, 3, 0, 2).reshape(BATCH, EMBED)
